# baseline (speedup 1.0000x reference)
import jax
import jax.numpy as jnp
from jax.experimental import pallas as pl


def _copy_kernel(w_ref, out_ref):
    out_ref[...] = w_ref[...]


def kernel(user_emb_weight):
    return pl.pallas_call(
        _copy_kernel,
        out_shape=jax.ShapeDtypeStruct((1, 128), jnp.float32),
    )(user_emb_weight)
